# Initial kernel scaffold; baseline (speedup 1.0000x reference)
import jax
import jax.numpy as jnp
from jax.experimental import pallas as pl


def kernel(input, W1, b1, Wpr, bpr, W2, b2, W3, b3):
    raise NotImplementedError("write your pallas kernel here")


# speedup vs baseline: 48.1768x; 48.1768x over previous
import functools

import jax
import jax.numpy as jnp
from jax import lax
from jax.experimental import pallas as pl

B = 512
D = 768
H1 = 128
FB = 8

_NEG = -3e38
_POS = 3e38


def _shr(a, k, fill):
    f = jnp.full(a.shape[:-1] + (k,), fill, a.dtype)
    return jnp.concatenate([f, a[..., :-k]], axis=-1)


def _shl(a, k, fill):
    if k == 0:
        return a
    f = jnp.full(a.shape[:-1] + (k,), fill, a.dtype)
    return jnp.concatenate([a[..., k:], f], axis=-1)


def _pr_block_kernel(w1_ref, b1_ref, x_ref, out_ref):
    w1 = w1_ref[...]
    x = x_ref[...]
    h = lax.dot_general(w1, x, (((1,), (1,)), ((), ())),
                        preferred_element_type=jnp.float32)
    h = jnp.maximum(h + b1_ref[...], 0.0)

    xi = h[:, :, None]
    xj = h[:, None, :]
    ii = lax.broadcasted_iota(jnp.int32, (FB, B, B), 1)
    jj = lax.broadcasted_iota(jnp.int32, (FB, B, B), 2)
    less = (xj < xi) | ((xj == xi) & (jj < ii))
    r = jnp.sum(less.astype(jnp.int32), axis=2)

    onehot = r[:, :, None] == jj
    s = jnp.sum(jnp.where(onehot, xi, 0.0), axis=1)

    inc = s
    sh = 1
    while sh < B:
        inc = inc + _shr(inc, sh, jnp.float32(0.0))
        sh *= 2

    best_cost = None
    best_sum = None
    for t in range(6):
        lo = s if t == 0 else _shr(s, t, _NEG)
        hi = s if t == 5 else _shl(s, 5 - t, _POS)
        cost = jnp.maximum(s - lo, hi - s)
        ia = inc if t == 5 else _shl(inc, 5 - t, jnp.float32(0.0))
        ib = _shr(inc, t + 1, jnp.float32(0.0))
        wsum = ia - ib
        if best_cost is None:
            best_cost, best_sum = cost, wsum
        else:
            take = cost < best_cost
            best_cost = jnp.where(take, cost, best_cost)
            best_sum = jnp.where(take, wsum, best_sum)
    m = best_sum * jnp.float32(1.0 / 6.0)

    out_ref[...] = jnp.sum(jnp.where(onehot, m[:, None, :], 0.0), axis=2)


def _dense_kernel(t_ref, wpr_ref, bpr_ref, w2_ref, b2_ref, w3_ref, b3_ref,
                  out_ref):
    t = t_ref[...]
    pr = lax.dot_general(wpr_ref[...], t, (((1,), (0,)), ((), ())),
                         preferred_element_type=jnp.float32)
    pr = jnp.maximum(pr + bpr_ref[...], 0.0)
    h2 = lax.dot_general(w2_ref[...], pr, (((1,), (0,)), ((), ())),
                         preferred_element_type=jnp.float32)
    h2 = jnp.maximum(h2 + b2_ref[...], 0.0)
    out = lax.dot_general(h2, w3_ref[...], (((0,), (1,)), ((), ())),
                          preferred_element_type=jnp.float32)
    out_ref[...] = out + b3_ref[...]


@jax.jit
def kernel(input, W1, b1, Wpr, bpr, W2, b2, W3, b3):
    nblk = H1 // FB
    t_T = pl.pallas_call(
        _pr_block_kernel,
        grid=(nblk,),
        in_specs=[
            pl.BlockSpec((FB, D), lambda i: (i, 0)),
            pl.BlockSpec((FB, 1), lambda i: (i, 0)),
            pl.BlockSpec((B, D), lambda i: (0, 0)),
        ],
        out_specs=pl.BlockSpec((FB, B), lambda i: (i, 0)),
        out_shape=jax.ShapeDtypeStruct((H1, B), jnp.float32),
    )(W1, b1.reshape(H1, 1), input)

    out = pl.pallas_call(
        _dense_kernel,
        out_shape=jax.ShapeDtypeStruct((B, W3.shape[0]), jnp.float32),
    )(t_T, Wpr, bpr.reshape(-1, 1), W2, b2.reshape(-1, 1), W3,
      b3.reshape(1, -1))
    return out


# speedup vs baseline: 56.9360x; 1.1818x over previous
import functools

import jax
import jax.numpy as jnp
from jax import lax
from jax.experimental import pallas as pl
from jax.experimental.pallas import tpu as pltpu
from jax.experimental.pallas import tpu_sc as plsc

B = 512
D = 768
H1 = 128
FB = 8
CPW = 4
ROWSTRIDE = 544
_NEG = -3e38
_POS = 3e38


def _rank_kernel(w1_ref, b1_ref, x_ref, h_ref, r_ref):
    w1 = w1_ref[...]
    x = x_ref[...]
    h = lax.dot_general(w1, x, (((1,), (1,)), ((), ())),
                        preferred_element_type=jnp.float32)
    h = jnp.maximum(h + b1_ref[...], 0.0)
    h_ref[...] = h

    xi = h[:, :, None]
    xj = h[:, None, :]
    ii = lax.broadcasted_iota(jnp.int32, (FB, B, B), 1)
    jj = lax.broadcasted_iota(jnp.int32, (FB, B, B), 2)
    less = (xj < xi) | ((xj == xi) & (jj < ii))
    r_ref[...] = jnp.sum(less.astype(jnp.int32), axis=2)


def _sc_transform(x_hbm, r_hbm, out_hbm, xv, rv, sp, mv, ov, sem):
    info = plsc.get_sparse_core_info()
    nc = info.num_cores
    wid = lax.axis_index("s") * nc + lax.axis_index("c")
    f0 = wid * CPW

    for c in range(CPW):
        pltpu.sync_copy(x_hbm.at[f0 + c], xv.at[pl.ds(c * B, B)])
        pltpu.sync_copy(r_hbm.at[f0 + c], rv.at[pl.ds(c * B, B)])

    def init_body(k, _):
        sp[pl.ds(k * 16, 16)] = jnp.full((16,), _POS, jnp.float32)
        return 0
    lax.fori_loop(0, CPW * ROWSTRIDE // 16, init_body, 0)
    lanes = lax.iota(jnp.int32, 16)
    edge = jnp.where(lanes >= 11, jnp.float32(_NEG), jnp.float32(_POS))
    for c in range(CPW):
        sp[pl.ds(c * ROWSTRIDE, 16)] = edge

    def scat_body(k, _):
        c = k // 32
        idx = rv[pl.ds(k * 16, 16)]
        val = xv[pl.ds(k * 16, 16)]
        plsc.store_scatter(sp, [idx + (c * ROWSTRIDE + 16)], val)
        return 0
    lax.fori_loop(0, CPW * 32, scat_body, 0)

    def win_body(k, _):
        c = k // 32
        kk = k - c * 32
        base = c * ROWSTRIDE + 16 + kk * 16
        s = [sp[pl.ds(base + d, 16)] for d in range(-5, 6)]
        x0 = s[5]
        wsum = s[5] + s[6] + s[7] + s[8] + s[9] + s[10]
        best_cost = jnp.maximum(x0 - s[5], s[10] - x0)
        best_sum = wsum
        for t in range(1, 6):
            wsum = wsum + s[5 - t] - s[11 - t]
            cost = jnp.maximum(x0 - s[5 - t], s[10 - t] - x0)
            take = cost < best_cost
            best_cost = jnp.where(take, cost, best_cost)
            best_sum = jnp.where(take, wsum, best_sum)
        mv[pl.ds(k * 16, 16)] = best_sum * jnp.float32(1.0 / 6.0)
        return 0
    lax.fori_loop(0, CPW * 32, win_body, 0)

    def gath_body(k, _):
        c = k // 32
        idx = rv[pl.ds(k * 16, 16)]
        ov[pl.ds(k * 16, 16)] = plsc.load_gather(mv, [idx + c * B])
        return 0
    lax.fori_loop(0, CPW * 32, gath_body, 0)

    for c in range(CPW):
        pltpu.sync_copy(ov.at[pl.ds(c * B, B)], out_hbm.at[f0 + c])


_sc_transform_call = functools.partial(
    pl.kernel,
    mesh=plsc.VectorSubcoreMesh(core_axis_name="c", subcore_axis_name="s"),
    out_type=jax.ShapeDtypeStruct((H1, B), jnp.float32),
    scratch_types=[
        pltpu.VMEM((CPW * B,), jnp.float32),
        pltpu.VMEM((CPW * B,), jnp.int32),
        pltpu.VMEM((CPW * ROWSTRIDE,), jnp.float32),
        pltpu.VMEM((CPW * B,), jnp.float32),
        pltpu.VMEM((CPW * B,), jnp.float32),
        pltpu.SemaphoreType.DMA,
    ],
    compiler_params=pltpu.CompilerParams(needs_layout_passes=False),
)(_sc_transform)


def _dense_kernel(t_ref, wpr_ref, bpr_ref, w2_ref, b2_ref, w3_ref, b3_ref,
                  out_ref):
    t = t_ref[...]
    pr = lax.dot_general(wpr_ref[...], t, (((1,), (0,)), ((), ())),
                         preferred_element_type=jnp.float32)
    pr = jnp.maximum(pr + bpr_ref[...], 0.0)
    h2 = lax.dot_general(w2_ref[...], pr, (((1,), (0,)), ((), ())),
                         preferred_element_type=jnp.float32)
    h2 = jnp.maximum(h2 + b2_ref[...], 0.0)
    out = lax.dot_general(h2, w3_ref[...], (((0,), (1,)), ((), ())),
                          preferred_element_type=jnp.float32)
    out_ref[...] = out + b3_ref[...]


@jax.jit
def kernel(input, W1, b1, Wpr, bpr, W2, b2, W3, b3):
    nblk = H1 // FB
    h_T, r_T = pl.pallas_call(
        _rank_kernel,
        grid=(nblk,),
        in_specs=[
            pl.BlockSpec((FB, D), lambda i: (i, 0)),
            pl.BlockSpec((FB, 1), lambda i: (i, 0)),
            pl.BlockSpec((B, D), lambda i: (0, 0)),
        ],
        out_specs=[
            pl.BlockSpec((FB, B), lambda i: (i, 0)),
            pl.BlockSpec((FB, B), lambda i: (i, 0)),
        ],
        out_shape=[
            jax.ShapeDtypeStruct((H1, B), jnp.float32),
            jax.ShapeDtypeStruct((H1, B), jnp.int32),
        ],
    )(W1, b1.reshape(H1, 1), input)

    t_T = _sc_transform_call(h_T, r_T)

    out = pl.pallas_call(
        _dense_kernel,
        out_shape=jax.ShapeDtypeStruct((B, W3.shape[0]), jnp.float32),
    )(t_T, Wpr, bpr.reshape(-1, 1), W2, b2.reshape(-1, 1), W3,
      b3.reshape(1, -1))
    return out


# speedup vs baseline: 113.6999x; 1.9970x over previous
import functools

import jax
import jax.numpy as jnp
from jax import lax
from jax.experimental import pallas as pl
from jax.experimental.pallas import tpu as pltpu
from jax.experimental.pallas import tpu_sc as plsc

B = 512
D = 768
H1 = 128
CPW = 4
_NEG = -3e38
_POS = 3e38


def _shr0(a, k, fill):
    f = jnp.full((k,) + a.shape[1:], fill, a.dtype)
    return jnp.concatenate([f, a[:-k]], axis=0)


def _shl0(a, k, fill):
    f = jnp.full((k,) + a.shape[1:], fill, a.dtype)
    return jnp.concatenate([a[k:], f], axis=0)


def _sort_kernel(w1_ref, b1_ref, x_ref, m_ref, ix_ref):
    h = lax.dot_general(x_ref[...], w1_ref[...], (((1,), (1,)), ((), ())),
                        preferred_element_type=jnp.float32)
    v = jnp.maximum(h + b1_ref[...], 0.0)

    ix = lax.broadcasted_iota(jnp.int32, (B, H1), 0)
    pidx = lax.broadcasted_iota(jnp.int32, (B, 1), 0)
    k = 2
    while k <= B:
        j = k // 2
        while j >= 1:
            mj = (pidx & j) != 0
            sm = ((pidx & k) == 0) != mj
            pv = jnp.where(mj, _shr0(v, j, 0.0), _shl0(v, j, 0.0))
            pi = jnp.where(mj, _shr0(ix, j, 0), _shl0(ix, j, 0))
            nv = jnp.where(sm, jnp.minimum(v, pv), jnp.maximum(v, pv))
            ix = jnp.where(nv == v, ix, pi)
            v = nv
            j //= 2
        k *= 2

    s = [_shr0(v, t, _NEG) for t in range(5, 0, -1)] + [v] + \
        [_shl0(v, t, _POS) for t in range(1, 6)]
    z = [_shr0(v, t, 0.0) for t in range(5, 0, -1)] + [v] + \
        [_shl0(v, t, 0.0) for t in range(1, 6)]
    x0 = v
    wsum = z[5] + z[6] + z[7] + z[8] + z[9] + z[10]
    best_cost = jnp.maximum(x0 - s[5], s[10] - x0)
    best_sum = wsum
    for t in range(1, 6):
        wsum = wsum + z[5 - t] - z[11 - t]
        cost = jnp.maximum(x0 - s[5 - t], s[10 - t] - x0)
        take = cost < best_cost
        best_cost = jnp.where(take, cost, best_cost)
        best_sum = jnp.where(take, wsum, best_sum)
    m = best_sum * jnp.float32(1.0 / 6.0)

    m_ref[...] = m.T
    ix_ref[...] = ix.T


def _sc_scatter(m_hbm, ix_hbm, out_hbm, mv, iv, ov, sem):
    info = plsc.get_sparse_core_info()
    nc = info.num_cores
    wid = lax.axis_index("s") * nc + lax.axis_index("c")
    f0 = wid * CPW

    for c in range(CPW):
        pltpu.sync_copy(m_hbm.at[f0 + c], mv.at[pl.ds(c * B, B)])
        pltpu.sync_copy(ix_hbm.at[f0 + c], iv.at[pl.ds(c * B, B)])

    def scat_body(kk, _):
        c = kk // 32
        idx = iv[pl.ds(kk * 16, 16)]
        val = mv[pl.ds(kk * 16, 16)]
        plsc.store_scatter(ov, [idx + c * B], val)
        return 0
    lax.fori_loop(0, CPW * 32, scat_body, 0)

    for c in range(CPW):
        pltpu.sync_copy(ov.at[pl.ds(c * B, B)], out_hbm.at[f0 + c])


def _sc_scatter_call(m_T, ix_T):
    call = pl.kernel(
        _sc_scatter,
        mesh=plsc.VectorSubcoreMesh(core_axis_name="c", subcore_axis_name="s"),
        out_type=jax.ShapeDtypeStruct((H1, B), jnp.float32),
        scratch_types=[
            pltpu.VMEM((CPW * B,), jnp.float32),
            pltpu.VMEM((CPW * B,), jnp.int32),
            pltpu.VMEM((CPW * B,), jnp.float32),
            pltpu.SemaphoreType.DMA,
        ],
        compiler_params=pltpu.CompilerParams(needs_layout_passes=False),
    )
    return call(m_T, ix_T)


def _dense_kernel(t_ref, wpr_ref, bpr_ref, w2_ref, b2_ref, w3_ref, b3_ref,
                  out_ref):
    t = t_ref[...]
    pr = lax.dot_general(wpr_ref[...], t, (((1,), (0,)), ((), ())),
                         preferred_element_type=jnp.float32)
    pr = jnp.maximum(pr + bpr_ref[...], 0.0)
    h2 = lax.dot_general(w2_ref[...], pr, (((1,), (0,)), ((), ())),
                         preferred_element_type=jnp.float32)
    h2 = jnp.maximum(h2 + b2_ref[...], 0.0)
    out = lax.dot_general(h2, w3_ref[...], (((0,), (1,)), ((), ())),
                          preferred_element_type=jnp.float32)
    out_ref[...] = out + b3_ref[...]


@jax.jit
def kernel(input, W1, b1, Wpr, bpr, W2, b2, W3, b3):
    m_T, ix_T = pl.pallas_call(
        _sort_kernel,
        out_shape=[
            jax.ShapeDtypeStruct((H1, B), jnp.float32),
            jax.ShapeDtypeStruct((H1, B), jnp.int32),
        ],
    )(W1, b1.reshape(1, H1), input)

    t_T = _sc_scatter_call(m_T, ix_T)

    out = pl.pallas_call(
        _dense_kernel,
        out_shape=jax.ShapeDtypeStruct((B, W3.shape[0]), jnp.float32),
    )(t_T, Wpr, bpr.reshape(-1, 1), W2, b2.reshape(-1, 1), W3,
      b3.reshape(1, -1))
    return out


# speedup vs baseline: 127.5652x; 1.1219x over previous
import functools

import jax
import jax.numpy as jnp
from jax import lax
from jax.experimental import pallas as pl
from jax.experimental.pallas import tpu as pltpu
from jax.experimental.pallas import tpu_sc as plsc

B = 512
D = 768
H1 = 128
CPW = 4
_NEG = -3e38
_POS = 3e38


def _shr0(a, k, fill):
    f = jnp.full((k,) + a.shape[1:], fill, a.dtype)
    return jnp.concatenate([f, a[:-k]], axis=0)


def _shl0(a, k, fill):
    f = jnp.full((k,) + a.shape[1:], fill, a.dtype)
    return jnp.concatenate([a[k:], f], axis=0)


def _sort_kernel(w1_ref, b1_ref, x_ref, m_ref, ix_ref):
    h = lax.dot_general(x_ref[...], w1_ref[...], (((1,), (1,)), ((), ())),
                        preferred_element_type=jnp.float32)
    v = jnp.maximum(h + b1_ref[...], 0.0)

    ix = lax.broadcasted_iota(jnp.int32, (B, H1), 0)
    pidx = lax.broadcasted_iota(jnp.int32, (B, 1), 0)
    k = 2
    while k <= B:
        j = k // 2
        while j >= 1:
            mj = (pidx & j) != 0
            sm = ((pidx & k) == 0) != mj
            pv = jnp.where(mj, _shr0(v, j, 0.0), _shl0(v, j, 0.0))
            pi = jnp.where(mj, _shr0(ix, j, 0), _shl0(ix, j, 0))
            nv = jnp.where(sm, jnp.minimum(v, pv), jnp.maximum(v, pv))
            ix = jnp.where(nv == v, ix, pi)
            v = nv
            j //= 2
        k *= 2

    s = [_shr0(v, t, _NEG) for t in range(5, 0, -1)] + [v] + \
        [_shl0(v, t, _POS) for t in range(1, 6)]
    z = [_shr0(v, t, 0.0) for t in range(5, 0, -1)] + [v] + \
        [_shl0(v, t, 0.0) for t in range(1, 6)]
    x0 = v
    wsum = z[5] + z[6] + z[7] + z[8] + z[9] + z[10]
    best_cost = jnp.maximum(x0 - s[5], s[10] - x0)
    best_sum = wsum
    for t in range(1, 6):
        wsum = wsum + z[5 - t] - z[11 - t]
        cost = jnp.maximum(x0 - s[5 - t], s[10 - t] - x0)
        take = cost < best_cost
        best_cost = jnp.where(take, cost, best_cost)
        best_sum = jnp.where(take, wsum, best_sum)
    m = best_sum * jnp.float32(1.0 / 6.0)

    m_ref[...] = m.T
    ix_ref[...] = ix.T


def _sc_scatter(m_hbm, ix_hbm, out_hbm, mv, iv, ov, sem):
    info = plsc.get_sparse_core_info()
    nc = info.num_cores
    wid = lax.axis_index("s") * nc + lax.axis_index("c")
    f0 = wid * CPW

    copies = []
    for c in range(CPW):
        copies.append(
            pltpu.async_copy(m_hbm.at[f0 + c], mv.at[pl.ds(c * B, B)], sem))
        copies.append(
            pltpu.async_copy(ix_hbm.at[f0 + c], iv.at[pl.ds(c * B, B)], sem))
    for cp in copies:
        cp.wait()

    def scat_body(kk, _):
        c = kk // 32
        idx = iv[pl.ds(kk * 16, 16)]
        val = mv[pl.ds(kk * 16, 16)]
        plsc.store_scatter(ov, [idx + c * B], val)
        return 0
    lax.fori_loop(0, CPW * 32, scat_body, 0, unroll=8)

    copies = [pltpu.async_copy(ov.at[pl.ds(c * B, B)], out_hbm.at[f0 + c], sem)
              for c in range(CPW)]
    for cp in copies:
        cp.wait()


def _sc_scatter_call(m_T, ix_T):
    call = pl.kernel(
        _sc_scatter,
        mesh=plsc.VectorSubcoreMesh(core_axis_name="c", subcore_axis_name="s"),
        out_type=jax.ShapeDtypeStruct((H1, B), jnp.float32),
        scratch_types=[
            pltpu.VMEM((CPW * B,), jnp.float32),
            pltpu.VMEM((CPW * B,), jnp.int32),
            pltpu.VMEM((CPW * B,), jnp.float32),
            pltpu.SemaphoreType.DMA,
        ],
        compiler_params=pltpu.CompilerParams(needs_layout_passes=False),
    )
    return call(m_T, ix_T)


def _dense_kernel(t_ref, wpr_ref, bpr_ref, w2_ref, b2_ref, w3_ref, b3_ref,
                  out_ref):
    t = t_ref[...]
    pr = lax.dot_general(wpr_ref[...], t, (((1,), (0,)), ((), ())),
                         preferred_element_type=jnp.float32)
    pr = jnp.maximum(pr + bpr_ref[...], 0.0)
    h2 = lax.dot_general(w2_ref[...], pr, (((1,), (0,)), ((), ())),
                         preferred_element_type=jnp.float32)
    h2 = jnp.maximum(h2 + b2_ref[...], 0.0)
    out = lax.dot_general(h2, w3_ref[...], (((0,), (1,)), ((), ())),
                          preferred_element_type=jnp.float32)
    out_ref[...] = out + b3_ref[...]


@jax.jit
def kernel(input, W1, b1, Wpr, bpr, W2, b2, W3, b3):
    m_T, ix_T = pl.pallas_call(
        _sort_kernel,
        out_shape=[
            jax.ShapeDtypeStruct((H1, B), jnp.float32),
            jax.ShapeDtypeStruct((H1, B), jnp.int32),
        ],
    )(W1, b1.reshape(1, H1), input)

    t_T = _sc_scatter_call(m_T, ix_T)

    out = pl.pallas_call(
        _dense_kernel,
        out_shape=jax.ShapeDtypeStruct((B, W3.shape[0]), jnp.float32),
    )(t_T, Wpr, bpr.reshape(-1, 1), W2, b2.reshape(-1, 1), W3,
      b3.reshape(1, -1))
    return out
